# baseline (speedup 1.0000x reference)
import functools

import jax
import jax.numpy as jnp
from jax import lax
from jax.experimental import pallas as pl
from jax.experimental.pallas import tpu as pltpu
from jax.experimental.pallas import tpu_sc as plsc

N_COMPOUND = 7000
N_PROTEIN = 3000
N_NODES = 10000
N_EDGES = 320000
H = 128
NEG = 0.01

NC, NS, L = 2, 16, 16
NW = NC * NS
EPW = N_EDGES // NW
NBKT = 79
BKTN = 128
NPAD = NBKT * BKTN
E_PAD = N_EDGES + NBKT * 128
E_ALL = E_PAD + 6144
NR = E_ALL // 128
TAIL = E_PAD
KW = 128
MROWS = 40
NCH = EPW // 128 + 1


def _leaky(x):
    return jnp.where(x >= 0, x, NEG * x)


def _worker_id():
    return lax.axis_index("s") * NC + lax.axis_index("c")


_LANE0 = None


def _lane0():
    return lax.broadcasted_iota(jnp.int32, (16,), 0) == 0


def _sstore(ref1d, index, value, dtype):
    plsc.store_scatter(ref1d, [jnp.full((16,), index, jnp.int32)],
                       jnp.full((16,), value, dtype), mask=_lane0())


def _sstore2(ref2d, row, col, value, dtype):
    plsc.store_scatter(ref2d,
                       [jnp.full((16,), row, jnp.int32),
                        jnp.full((16,), col, jnp.int32)],
                       jnp.full((16,), value, dtype), mask=_lane0())


_sc_mesh = plsc.VectorSubcoreMesh(core_axis_name="c", subcore_axis_name="s")


@functools.partial(
    pl.kernel, mesh=_sc_mesh,
    compiler_params=pltpu.CompilerParams(needs_layout_passes=False),
    out_type=jax.ShapeDtypeStruct((NW, 80), jnp.int32),
    scratch_types=[pltpu.VMEM((EPW,), jnp.int32), pltpu.VMEM((80,), jnp.int32),
                   pltpu.SMEM((80,), jnp.int32)],
)
def _sc_histogram(dst_hbm, cnt_hbm, dstv, histv, hists):
    wid = _worker_id()
    pltpu.sync_copy(dst_hbm.at[pl.ds(pl.multiple_of(wid * EPW, 8), EPW)], dstv)

    def zero(i, _):
        hists[i] = 0
        return 0
    lax.fori_loop(0, 80, zero, 0)

    def count(g, _):
        bvec = lax.shift_right_logical(dstv[pl.ds(g * 16, 16)], 7)
        for k in range(16):
            b = bvec[k]
            hists[b] = hists[b] + 1
        return 0
    lax.fori_loop(0, EPW // 16, count, 0)

    def tovmem(i, _):
        _sstore(histv, i, hists[i], jnp.int32)
        return 0
    lax.fori_loop(0, 80, tovmem, 0)
    pltpu.sync_copy(histv, cnt_hbm.at[wid])


@functools.partial(
    pl.kernel, mesh=_sc_mesh,
    compiler_params=pltpu.CompilerParams(needs_layout_passes=False),
    out_type=(jax.ShapeDtypeStruct((E_ALL,), jnp.int32),
              jax.ShapeDtypeStruct((E_ALL,), jnp.float32),
              jax.ShapeDtypeStruct((E_ALL,), jnp.int32),
              jax.ShapeDtypeStruct((E_ALL,), jnp.int32),
              jax.ShapeDtypeStruct((88,), jnp.int32)),
    scratch_types=[
        pltpu.VMEM((NCH * 128,), jnp.int32),
        pltpu.VMEM((NCH * 128,), jnp.float32),
        pltpu.VMEM((NCH * 128,), jnp.int32),
        pltpu.VMEM((NCH * 128,), jnp.int32),
        pltpu.VMEM((NW, 80), jnp.int32),
        pltpu.VMEM((88,), jnp.int32),
        pltpu.SMEM((88,), jnp.int32),
        pltpu.SMEM((80,), jnp.int32),
        pltpu.SMEM((80,), jnp.int32),
        pltpu.VMEM((NCH, 128), jnp.int32),
        pltpu.VMEM((3, 128), jnp.int32),
        pltpu.VMEM((3, 128), jnp.int32),
        pltpu.VMEM((128,), jnp.int32),
        pltpu.VMEM((128,), jnp.float32),
        pltpu.SemaphoreType.DMA,
    ],
)
def _sc_permute(src_hbm, wgt_hbm, type_hbm, dst_hbm, cnt_hbm,
                srcg_hbm, wgtg_hbm, typeg_hbm, dstg_hbm, bs_hbm,
                srcv, wgtv, typev, dstv, cntv, startv, starts, curs, tots,
                pos2d, padpos, paddst, zeros_i, zeros_f, sem):
    wid = _worker_id()
    pltpu.sync_copy(cnt_hbm, cntv)
    pltpu.sync_copy(src_hbm.at[pl.ds(pl.multiple_of(wid * EPW, 8), EPW)], srcv.at[pl.ds(0, EPW)])
    pltpu.sync_copy(wgt_hbm.at[pl.ds(pl.multiple_of(wid * EPW, 8), EPW)], wgtv.at[pl.ds(0, EPW)])
    pltpu.sync_copy(type_hbm.at[pl.ds(pl.multiple_of(wid * EPW, 8), EPW)], typev.at[pl.ds(0, EPW)])
    pltpu.sync_copy(dst_hbm.at[pl.ds(pl.multiple_of(wid * EPW, 8), EPW)], dstv.at[pl.ds(0, EPW)])

    zero16 = jnp.zeros((16,), jnp.int32)
    tot_chunks = []
    mine_chunks = []
    for j in range(5):
        sl = pl.ds(j * 16, 16)

        def addall(t, acc, sl=sl):
            return acc + cntv[t, sl]
        tot_chunks.append(lax.fori_loop(0, NW, addall, zero16))
        mine_chunks.append(lax.fori_loop(0, wid, addall, zero16))

    acc = jnp.int32(0)
    for b in range(NBKT):
        tot_b = tot_chunks[b // 16][b % 16]
        mine_b = mine_chunks[b // 16][b % 16]
        starts[b] = acc
        curs[b] = acc + mine_b
        tots[b] = tot_b
        acc = acc + lax.bitwise_and(tot_b + 127, -128)
    for b in range(NBKT, 88):
        starts[b] = acc

    def starts_tovmem(i, _):
        _sstore(startv, i, starts[i], jnp.int32)
        return 0
    lax.fori_loop(0, 88, starts_tovmem, 0)

    @pl.when(wid == 0)
    def _():
        pltpu.sync_copy(startv, bs_hbm)

    z16i = jnp.zeros((16,), jnp.int32)
    z16f = jnp.zeros((16,), jnp.float32)
    iota16c = lax.broadcasted_iota(jnp.int32, (16,), 0)
    for j in range(8):
        sl = pl.ds(j * 16, 16)
        zeros_i[sl] = z16i
        zeros_f[sl] = z16f
    for c in range(3):
        for j in range(8):
            sl = pl.ds(j * 16, 16)
            padpos[c, sl] = jnp.full((16,), TAIL + 512, jnp.int32) + \
                iota16c + (c * 128 + j * 16)
            paddst[c, sl] = z16i

    iota16 = lax.broadcasted_iota(jnp.int32, (16,), 0)
    for j in range(1, 8):
        pos2d[NCH - 1, pl.ds(j * 16, 16)] = TAIL + iota16 + j * 16

    def rank(g, _):
        gb = g * 16
        bvec = lax.shift_right_logical(dstv[pl.ds(gb, 16)], 7)
        row = lax.shift_right_logical(gb, 7)
        col0 = lax.bitwise_and(gb, 127)
        acc = zero16
        for k in range(16):
            b = bvec[k]
            p = curs[b]
            curs[b] = p + 1
            acc = jnp.where(iota16 == k, p, acc)
        pos2d[row, pl.ds(col0, 16)] = acc
        return 0
    lax.fori_loop(0, EPW // 16, rank, 0)

    def fill_bucket(rnd, slot):
        b = wid + rnd * NW
        b_c = jnp.minimum(b, NBKT - 1)
        gapstart = starts[b_c] + tots[b_c]
        gap = jnp.where(b < NBKT, starts[b_c + 1] - gapstart, 0)

        def put(g, slot):
            _sstore2(padpos, lax.shift_right_logical(slot, 7),
                     lax.bitwise_and(slot, 127), gapstart + g, jnp.int32)
            _sstore2(paddst, lax.shift_right_logical(slot, 7),
                     lax.bitwise_and(slot, 127), b_c * BKTN + BKTN, jnp.int32)
            return slot + 1
        return lax.fori_loop(0, gap, put, slot)
    lax.fori_loop(0, 3, fill_bucket, 0)

    pend = []
    for j in range(NCH):
        sl = pl.ds(j * 128, 128)
        idx = pos2d.at[j]
        pend.append(pltpu.async_copy(srcv.at[sl], srcg_hbm.at[idx], sem))
        pend.append(pltpu.async_copy(wgtv.at[sl], wgtg_hbm.at[idx], sem))
        pend.append(pltpu.async_copy(typev.at[sl], typeg_hbm.at[idx], sem))
        pend.append(pltpu.async_copy(dstv.at[sl], dstg_hbm.at[idx], sem))
        if len(pend) >= 16:
            for cp in pend:
                cp.wait()
            pend = []
    for c in range(3):
        idx = padpos.at[c]
        pend.append(pltpu.async_copy(zeros_i, srcg_hbm.at[idx], sem))
        pend.append(pltpu.async_copy(zeros_f, wgtg_hbm.at[idx], sem))
        pend.append(pltpu.async_copy(zeros_i, typeg_hbm.at[idx], sem))
        pend.append(pltpu.async_copy(paddst.at[c], dstg_hbm.at[idx], sem))
    for cp in pend:
        cp.wait()


@functools.partial(
    pl.kernel, mesh=_sc_mesh,
    compiler_params=pltpu.CompilerParams(needs_layout_passes=False),
    out_type=(jax.ShapeDtypeStruct((NPAD, H), jnp.float32),
              jax.ShapeDtypeStruct((NPAD, H), jnp.float32),
              jax.ShapeDtypeStruct((NPAD, H), jnp.float32),
              jax.ShapeDtypeStruct((NPAD, H), jnp.float32),
              jax.ShapeDtypeStruct((NPAD,), jnp.float32)),
    scratch_types=[
        pltpu.VMEM((136, H), jnp.float32),
        pltpu.VMEM((136, H), jnp.float32),
        pltpu.VMEM((136,), jnp.float32),
        pltpu.SMEM((136,), jnp.float32),
        pltpu.VMEM((KW, H), jnp.float32),
        pltpu.VMEM((MROWS + 8, 128), jnp.int32),
        pltpu.VMEM((MROWS + 8, 128), jnp.float32),
        pltpu.VMEM((MROWS + 8, 128), jnp.int32),
        pltpu.VMEM((MROWS + 8, 128), jnp.int32),
        pltpu.VMEM((4, H), jnp.float32),
        pltpu.VMEM((96,), jnp.int32),
        pltpu.SemaphoreType.DMA,
        pltpu.SemaphoreType.DMA,
    ] + [pltpu.VMEM((136 * 16,), jnp.float32) for _ in range(16)],
)
def _sc_stats(b_hbm, t_hbm, srcg_hbm, wgtg_hbm, typeg_hbm, dstg_hbm, bs_hbm,
              s1_hbm, s2_hbm, mx_hbm, mn_hbm, deg_hbm,
              s1a, s2a, degav, dega, rowsv,
              src2d, wgt2d, ty2d, dst2d, tv, startv, sem0, sem1, *mmrefs):
    mx8 = mmrefs[0:8]
    mn8 = mmrefs[8:16]
    wid = _worker_id()
    pltpu.sync_copy(bs_hbm, startv.at[pl.ds(0, 88)])
    pltpu.sync_copy(t_hbm, tv)

    z16 = jnp.zeros((16,), jnp.float32)
    lo16 = jnp.full((16,), -1e30, jnp.float32)
    hi16 = jnp.full((16,), 1e30, jnp.float32)

    def do_round(rnd, _):
        b = wid + rnd * NW

        @pl.when(b < NBKT)
        def _process():
            base = b * BKTN
            sv = startv[pl.ds(b, 16)]
            start = sv[0]
            n_e = sv[1] - sv[0]
            nwin = lax.shift_right_logical(n_e, 7)
            r0 = lax.shift_right_logical(start, 7)
            roff = lax.bitwise_and(r0, 7)
            r0a = pl.multiple_of(r0 - roff, 8)

            pltpu.sync_copy(srcg_hbm.at[pl.ds(r0a, MROWS + 8)], src2d)
            pltpu.sync_copy(wgtg_hbm.at[pl.ds(r0a, MROWS + 8)], wgt2d)
            pltpu.sync_copy(typeg_hbm.at[pl.ds(r0a, MROWS + 8)], ty2d)
            pltpu.sync_copy(dstg_hbm.at[pl.ds(r0a, MROWS + 8)], dst2d)

            def zero(d, _):
                for j in range(8):
                    sl = pl.ds(j * 16, 16)
                    s1a[d, sl] = z16
                    s2a[d, sl] = z16
                    mx8[j][pl.ds(d * 16, 16)] = lo16
                    mn8[j][pl.ds(d * 16, 16)] = hi16
                dega[d] = jnp.float32(0.0)
                return 0
            lax.fori_loop(0, 136, zero, 0)

            def window(w, _):
                pltpu.async_copy(b_hbm.at[src2d.at[roff + w]],
                                 rowsv, sem0).wait()

                def group(gq, _):
                    sl16 = pl.ds(gq * 16, 16)
                    dvec = dst2d[roff + w, sl16] - base
                    wvec = wgt2d[roff + w, sl16]
                    tyvec = ty2d[roff + w, sl16]
                    for k in range(16):
                        e = gq * 16 + k
                        d = dvec[k]
                        ty = tyvec[k]
                        wbc = wvec.at[jnp.full((16,), k, jnp.int32)].get(
                            mode="promise_in_bounds")
                        dega[d] = dega[d] + 1.0
                        for j in range(8):
                            sl = pl.ds(j * 16, 16)
                            r = rowsv[e, sl] + wbc * tv[ty, sl]
                            plsc.addupdate(s1a.at[d, sl], r)
                            plsc.addupdate(s2a.at[d, sl], r * r)
                            dsl16 = pl.ds(d * 16, 16)
                            mx8[j][dsl16] = jnp.maximum(mx8[j][dsl16], r)
                            mn8[j][dsl16] = jnp.minimum(mn8[j][dsl16], r)
                    return 0
                lax.fori_loop(0, 8, group, 0)
                return 0
            lax.fori_loop(0, nwin, window, 0)

            def deg_tovmem(d, _):
                _sstore(degav, d, dega[d], jnp.float32)
                return 0
            lax.fori_loop(0, BKTN, deg_tovmem, 0)

            dsl = pl.ds(0, BKTN)
            osl = pl.ds(pl.multiple_of(base, BKTN), BKTN)
            pltpu.sync_copy(s1a.at[dsl], s1_hbm.at[osl])
            pltpu.sync_copy(s2a.at[dsl], s2_hbm.at[osl])

            def gathermx(d, _):
                for j in range(8):
                    s1a[d, pl.ds(j * 16, 16)] = mx8[j][pl.ds(d * 16, 16)]
                return 0
            lax.fori_loop(0, BKTN, gathermx, 0)
            pltpu.sync_copy(s1a.at[dsl], mx_hbm.at[osl])

            def gathermn(d, _):
                for j in range(8):
                    s1a[d, pl.ds(j * 16, 16)] = mn8[j][pl.ds(d * 16, 16)]
                return 0
            lax.fori_loop(0, BKTN, gathermn, 0)
            pltpu.sync_copy(s1a.at[dsl], mn_hbm.at[osl])
            pltpu.sync_copy(degav.at[dsl], deg_hbm.at[osl])
        return 0
    lax.fori_loop(0, 3, do_round, 0)


def _mm_bias_body(x_ref, w_ref, b_ref, o_ref):
    o_ref[...] = x_ref[...] @ w_ref[...] + b_ref[...]


def _mm_bias(x, w, b, bm):
    m = x.shape[0]
    return pl.pallas_call(
        _mm_bias_body,
        grid=(m // bm,),
        in_specs=[pl.BlockSpec((bm, x.shape[1]), lambda i: (i, 0)),
                  pl.BlockSpec(w.shape, lambda i: (0, 0)),
                  pl.BlockSpec((1, w.shape[1]), lambda i: (0, 0))],
        out_specs=pl.BlockSpec((bm, w.shape[1]), lambda i: (i, 0)),
        out_shape=jax.ShapeDtypeStruct((m, w.shape[1]), jnp.float32),
    )(x, w, b[None, :])


def _ab_body(x_ref, wpre_ref, wedge_ref, eemb_ref, bedge_ref, bpre_ref,
             a_ref, b_ref, t_ref):
    w1 = wpre_ref[0:H, :]
    w2 = wpre_ref[H:2 * H, :]
    w3 = wpre_ref[2 * H:3 * H, :]
    x = x_ref[...]
    a_ref[...] = x @ w1
    c0 = bedge_ref[...] @ w3 + bpre_ref[...]
    b_ref[...] = x @ w2 + c0
    t_ref[...] = (eemb_ref[...] @ wedge_ref[...]) @ w3


def _ab_stage(entity, lp, eemb):
    bm = 1000
    return pl.pallas_call(
        _ab_body,
        grid=(N_NODES // bm,),
        in_specs=[pl.BlockSpec((bm, H), lambda i: (i, 0)),
                  pl.BlockSpec((3 * H, H), lambda i: (0, 0)),
                  pl.BlockSpec((16, H), lambda i: (0, 0)),
                  pl.BlockSpec((4, 16), lambda i: (0, 0)),
                  pl.BlockSpec((1, H), lambda i: (0, 0)),
                  pl.BlockSpec((1, H), lambda i: (0, 0))],
        out_specs=(pl.BlockSpec((bm, H), lambda i: (i, 0)),
                   pl.BlockSpec((bm, H), lambda i: (i, 0)),
                   pl.BlockSpec((4, H), lambda i: (0, 0))),
        out_shape=(jax.ShapeDtypeStruct((N_NODES, H), jnp.float32),
                   jax.ShapeDtypeStruct((N_NODES, H), jnp.float32),
                   jax.ShapeDtypeStruct((4, H), jnp.float32)),
    )(entity, lp['W_pre'], lp['W_edge'], eemb, lp['b_edge'][None, :],
      lp['b_pre'][None, :])


def _post_body(x_ref, a_ref, s1_ref, s2_ref, mx_ref, mn_ref, deg_ref,
               adl_ref, wpost_ref, bpost_ref, wlin_ref, blin_ref, o_ref):
    deg = deg_ref[...]
    degc = jnp.maximum(deg, 1.0)
    idegc = 1.0 / degc
    has = deg > 0
    a = a_ref[...]
    s1n = s1_ref[...] * idegc
    mean = jnp.where(has, a + s1n, 0.0)
    std = jnp.sqrt(jax.nn.relu(s2_ref[...] * idegc - s1n * s1n) + 1e-5)
    mx = jnp.where(has, a + mx_ref[...], 0.0)
    mn = jnp.where(has, a + mn_ref[...], 0.0)
    agg = jnp.concatenate([mean, mn, mx, std], axis=-1)
    adl = adl_ref[0, 0]
    lg = jnp.log(degc + 1.0)
    amp = lg * (1.0 / adl)
    att = adl / lg
    cat = jnp.concatenate([x_ref[...], agg, agg * amp, agg * att], axis=-1)
    out = cat @ wpost_ref[...] + bpost_ref[...]
    out = out @ wlin_ref[...] + blin_ref[...]
    o_ref[...] = _leaky(out)


def _post_stage(entity, a, s1, s2, mx, mn, deg, lp):
    bm = 1000
    nb = N_NODES // bm
    row = lambda i: (i, 0)
    fixed = lambda i: (0, 0)
    return pl.pallas_call(
        _post_body,
        grid=(nb,),
        in_specs=[pl.BlockSpec((bm, H), row),
                  pl.BlockSpec((bm, H), row),
                  pl.BlockSpec((bm, H), row),
                  pl.BlockSpec((bm, H), row),
                  pl.BlockSpec((bm, H), row),
                  pl.BlockSpec((bm, H), row),
                  pl.BlockSpec((bm, 1), row),
                  pl.BlockSpec((1, 1), fixed),
                  pl.BlockSpec((13 * H, H), fixed),
                  pl.BlockSpec((1, H), fixed),
                  pl.BlockSpec((H, H), fixed),
                  pl.BlockSpec((1, H), fixed)],
        out_specs=pl.BlockSpec((bm, H), row),
        out_shape=jax.ShapeDtypeStruct((N_NODES, H), jnp.float32),
    )(entity, a, s1, s2, mx, mn, deg[:, None],
      jnp.reshape(lp['avg_deg_log'], (1, 1)), lp['W_post'],
      lp['b_post'][None, :], lp['W_lin'], lp['b_lin'][None, :])


def _heads_body(comp_ref, prot_ref, wfc_ref, bfc_ref, wfp_ref, bfp_ref,
                w0_ref, w1_ref, u_ref, v_ref):
    fc = _leaky(comp_ref[...] @ wfc_ref[...] + bfc_ref[...])
    fc = fc / jnp.clip(jnp.sqrt(jnp.sum(fc * fc, axis=1, keepdims=True)), 1e-12)
    u_ref[...] = fc @ w0_ref[...]
    fp = _leaky(prot_ref[...] @ wfp_ref[...] + bfp_ref[...])
    fp = fp / jnp.clip(jnp.sqrt(jnp.sum(fp * fp, axis=1, keepdims=True)), 1e-12)
    v_ref[...] = fp @ w1_ref[...]


def _final_stage(comp, prot, params):
    return pl.pallas_call(
        _heads_body,
        out_shape=(jax.ShapeDtypeStruct((N_COMPOUND, 64), jnp.float32),
                   jax.ShapeDtypeStruct((N_PROTEIN, 64), jnp.float32)),
    )(comp, prot, params['Wfc'], params['bfc'][None, :], params['Wfp'],
      params['bfp'][None, :], params['W0p'], params['W1p'])


def _recon_body(u_ref, v_ref, cpi_ref, mask_ref, recon_ref, loss_ref):
    i = pl.program_id(0)
    j = pl.program_id(1)
    recon = u_ref[...] @ v_ref[...].T
    recon_ref[...] = recon
    tmp = mask_ref[...] * (recon - cpi_ref[...])
    part = jnp.sum(tmp * tmp)

    @pl.when((i == 0) & (j == 0))
    def _init():
        loss_ref[0, 0] = 0.0

    loss_ref[0, 0] += part


def _recon_loss(u, v, CPI, CPI_mask):
    bm, bn = 280, 3000
    grid = (N_COMPOUND // bm, N_PROTEIN // bn)
    recon, loss = pl.pallas_call(
        _recon_body,
        grid=grid,
        in_specs=[
            pl.BlockSpec((bm, 64), lambda i, j: (i, 0)),
            pl.BlockSpec((bn, 64), lambda i, j: (j, 0)),
            pl.BlockSpec((bm, bn), lambda i, j: (i, j)),
            pl.BlockSpec((bm, bn), lambda i, j: (i, j)),
        ],
        out_specs=(pl.BlockSpec((bm, bn), lambda i, j: (i, j)),
                   pl.BlockSpec((1, 1), lambda i, j: (0, 0),
                                memory_space=pltpu.SMEM)),
        out_shape=(jax.ShapeDtypeStruct((N_COMPOUND, N_PROTEIN), jnp.float32),
                   jax.ShapeDtypeStruct((1, 1), jnp.float32)),
    )(u, v, CPI, CPI_mask)
    return loss[0, 0], recon


def kernel(compound_embedding, protein_embedding, edge_index, edge_weight,
           edge_type, CPI, CPI_mask, params):
    src = edge_index[0]
    dst = edge_index[1]
    ew = edge_weight[:, 0]
    etype = edge_type.astype(jnp.int32)

    compound = _mm_bias(compound_embedding, params['Wc'], params['bc'], 1000)
    protein = _mm_bias(protein_embedding, params['Wp'], params['bp'], 1000)
    entity = jnp.concatenate([compound, protein], axis=0)
    entity0 = entity

    cnt = _sc_histogram(dst)
    src_g, wgt_g, type_g, dst_g, bkt_start = _sc_permute(src, ew, etype,
                                                         dst, cnt)
    src_g = src_g.reshape(NR, 128)
    wgt_g = wgt_g.reshape(NR, 128)
    type_g = type_g.reshape(NR, 128)
    dst_g = dst_g.reshape(NR, 128)

    for lp in params['layers']:
        a, b_eff, t_tab = _ab_stage(entity, lp, params['edge_emb'])
        s1, s2, mx, mn, deg = _sc_stats(b_eff, t_tab, src_g, wgt_g, type_g,
                                        dst_g, bkt_start)
        entity = _post_stage(entity, a, s1[:N_NODES], s2[:N_NODES],
                             mx[:N_NODES], mn[:N_NODES], deg[:N_NODES], lp)

    entity = jnp.concatenate([entity, entity0], axis=1)
    comp = entity[:N_COMPOUND]
    prot = entity[N_COMPOUND:N_COMPOUND + N_PROTEIN]
    u, v = _final_stage(comp, prot, params)
    return _recon_loss(u, v, CPI, CPI_mask)


# speedup vs baseline: 1.1679x; 1.1679x over previous
import functools

import jax
import jax.numpy as jnp
from jax import lax
from jax.experimental import pallas as pl
from jax.experimental.pallas import tpu as pltpu
from jax.experimental.pallas import tpu_sc as plsc

N_COMPOUND = 7000
N_PROTEIN = 3000
N_NODES = 10000
N_EDGES = 320000
H = 128
NEG = 0.01

NC, NS, L = 2, 16, 16
NW = NC * NS
EPW = N_EDGES // NW
NBKT = 79
BKTN = 128
NPAD = NBKT * BKTN
E_PAD = N_EDGES + NBKT * 128
E_ALL = E_PAD + 6144
NR = E_ALL // 128
TAIL = E_PAD
KW = 128
MROWS = 40
NCH = EPW // 128 + 1


def _leaky(x):
    return jnp.where(x >= 0, x, NEG * x)


def _worker_id():
    return lax.axis_index("s") * NC + lax.axis_index("c")


_LANE0 = None


def _lane0():
    return lax.broadcasted_iota(jnp.int32, (16,), 0) == 0


def _sstore(ref1d, index, value, dtype):
    plsc.store_scatter(ref1d, [jnp.full((16,), index, jnp.int32)],
                       jnp.full((16,), value, dtype), mask=_lane0())


def _sstore2(ref2d, row, col, value, dtype):
    plsc.store_scatter(ref2d,
                       [jnp.full((16,), row, jnp.int32),
                        jnp.full((16,), col, jnp.int32)],
                       jnp.full((16,), value, dtype), mask=_lane0())


_sc_mesh = plsc.VectorSubcoreMesh(core_axis_name="c", subcore_axis_name="s")


@functools.partial(
    pl.kernel, mesh=_sc_mesh,
    compiler_params=pltpu.CompilerParams(needs_layout_passes=False),
    out_type=jax.ShapeDtypeStruct((NW, 80), jnp.int32),
    scratch_types=[pltpu.VMEM((EPW,), jnp.int32), pltpu.VMEM((80,), jnp.int32),
                   pltpu.SMEM((80,), jnp.int32)],
)
def _sc_histogram(dst_hbm, cnt_hbm, dstv, histv, hists):
    wid = _worker_id()
    pltpu.sync_copy(dst_hbm.at[pl.ds(pl.multiple_of(wid * EPW, 8), EPW)], dstv)

    def zero(i, _):
        hists[i] = 0
        return 0
    lax.fori_loop(0, 80, zero, 0)

    def count(g, _):
        bvec = lax.shift_right_logical(dstv[pl.ds(g * 16, 16)], 7)
        for k in range(16):
            b = bvec[k]
            hists[b] = hists[b] + 1
        return 0
    lax.fori_loop(0, EPW // 16, count, 0)

    def tovmem(i, _):
        _sstore(histv, i, hists[i], jnp.int32)
        return 0
    lax.fori_loop(0, 80, tovmem, 0)
    pltpu.sync_copy(histv, cnt_hbm.at[wid])


@functools.partial(
    pl.kernel, mesh=_sc_mesh,
    compiler_params=pltpu.CompilerParams(needs_layout_passes=False),
    out_type=(jax.ShapeDtypeStruct((E_ALL,), jnp.int32),
              jax.ShapeDtypeStruct((E_ALL,), jnp.float32),
              jax.ShapeDtypeStruct((E_ALL,), jnp.int32),
              jax.ShapeDtypeStruct((E_ALL,), jnp.int32),
              jax.ShapeDtypeStruct((88,), jnp.int32)),
    scratch_types=[
        pltpu.VMEM((NCH * 128,), jnp.int32),
        pltpu.VMEM((NCH * 128,), jnp.float32),
        pltpu.VMEM((NCH * 128,), jnp.int32),
        pltpu.VMEM((NCH * 128,), jnp.int32),
        pltpu.VMEM((NW, 80), jnp.int32),
        pltpu.VMEM((88,), jnp.int32),
        pltpu.SMEM((88,), jnp.int32),
        pltpu.SMEM((80,), jnp.int32),
        pltpu.SMEM((80,), jnp.int32),
        pltpu.VMEM((NCH, 128), jnp.int32),
        pltpu.VMEM((3, 128), jnp.int32),
        pltpu.VMEM((3, 128), jnp.int32),
        pltpu.VMEM((128,), jnp.int32),
        pltpu.VMEM((128,), jnp.float32),
        pltpu.SemaphoreType.DMA,
    ],
)
def _sc_permute(src_hbm, wgt_hbm, type_hbm, dst_hbm, cnt_hbm,
                srcg_hbm, wgtg_hbm, typeg_hbm, dstg_hbm, bs_hbm,
                srcv, wgtv, typev, dstv, cntv, startv, starts, curs, tots,
                pos2d, padpos, paddst, zeros_i, zeros_f, sem):
    wid = _worker_id()
    pltpu.sync_copy(cnt_hbm, cntv)
    pltpu.sync_copy(src_hbm.at[pl.ds(pl.multiple_of(wid * EPW, 8), EPW)], srcv.at[pl.ds(0, EPW)])
    pltpu.sync_copy(wgt_hbm.at[pl.ds(pl.multiple_of(wid * EPW, 8), EPW)], wgtv.at[pl.ds(0, EPW)])
    pltpu.sync_copy(type_hbm.at[pl.ds(pl.multiple_of(wid * EPW, 8), EPW)], typev.at[pl.ds(0, EPW)])
    pltpu.sync_copy(dst_hbm.at[pl.ds(pl.multiple_of(wid * EPW, 8), EPW)], dstv.at[pl.ds(0, EPW)])

    zero16 = jnp.zeros((16,), jnp.int32)
    tot_chunks = []
    mine_chunks = []
    for j in range(5):
        sl = pl.ds(j * 16, 16)

        def addall(t, acc, sl=sl):
            return acc + cntv[t, sl]
        tot_chunks.append(lax.fori_loop(0, NW, addall, zero16))
        mine_chunks.append(lax.fori_loop(0, wid, addall, zero16))

    acc = jnp.int32(0)
    for b in range(NBKT):
        tot_b = tot_chunks[b // 16][b % 16]
        mine_b = mine_chunks[b // 16][b % 16]
        starts[b] = acc
        curs[b] = acc + mine_b
        tots[b] = tot_b
        acc = acc + lax.bitwise_and(tot_b + 127, -128)
    for b in range(NBKT, 88):
        starts[b] = acc

    def starts_tovmem(i, _):
        _sstore(startv, i, starts[i], jnp.int32)
        return 0
    lax.fori_loop(0, 88, starts_tovmem, 0)

    @pl.when(wid == 0)
    def _():
        pltpu.sync_copy(startv, bs_hbm)

    z16i = jnp.zeros((16,), jnp.int32)
    z16f = jnp.zeros((16,), jnp.float32)
    iota16c = lax.broadcasted_iota(jnp.int32, (16,), 0)
    for j in range(8):
        sl = pl.ds(j * 16, 16)
        zeros_i[sl] = z16i
        zeros_f[sl] = z16f
    for c in range(3):
        for j in range(8):
            sl = pl.ds(j * 16, 16)
            padpos[c, sl] = jnp.full((16,), TAIL + 512, jnp.int32) + \
                iota16c + (c * 128 + j * 16)
            paddst[c, sl] = z16i

    iota16 = lax.broadcasted_iota(jnp.int32, (16,), 0)
    for j in range(1, 8):
        pos2d[NCH - 1, pl.ds(j * 16, 16)] = TAIL + iota16 + j * 16

    def rank(g, _):
        gb = g * 16
        bvec = lax.shift_right_logical(dstv[pl.ds(gb, 16)], 7)
        row = lax.shift_right_logical(gb, 7)
        col0 = lax.bitwise_and(gb, 127)
        acc = zero16
        for k in range(16):
            b = bvec[k]
            p = curs[b]
            curs[b] = p + 1
            acc = jnp.where(iota16 == k, p, acc)
        pos2d[row, pl.ds(col0, 16)] = acc
        return 0
    lax.fori_loop(0, EPW // 16, rank, 0)

    def fill_bucket(rnd, slot):
        b = wid + rnd * NW
        b_c = jnp.minimum(b, NBKT - 1)
        gapstart = starts[b_c] + tots[b_c]
        gap = jnp.where(b < NBKT, starts[b_c + 1] - gapstart, 0)

        def put(g, slot):
            _sstore2(padpos, lax.shift_right_logical(slot, 7),
                     lax.bitwise_and(slot, 127), gapstart + g, jnp.int32)
            _sstore2(paddst, lax.shift_right_logical(slot, 7),
                     lax.bitwise_and(slot, 127), b_c * BKTN + BKTN, jnp.int32)
            return slot + 1
        return lax.fori_loop(0, gap, put, slot)
    lax.fori_loop(0, 3, fill_bucket, 0)

    pend = []
    for j in range(NCH):
        sl = pl.ds(j * 128, 128)
        idx = pos2d.at[j]
        pend.append(pltpu.async_copy(srcv.at[sl], srcg_hbm.at[idx], sem))
        pend.append(pltpu.async_copy(wgtv.at[sl], wgtg_hbm.at[idx], sem))
        pend.append(pltpu.async_copy(typev.at[sl], typeg_hbm.at[idx], sem))
        pend.append(pltpu.async_copy(dstv.at[sl], dstg_hbm.at[idx], sem))
        if len(pend) >= 64:
            for cp in pend:
                cp.wait()
            pend = []
    for c in range(3):
        idx = padpos.at[c]
        pend.append(pltpu.async_copy(zeros_i, srcg_hbm.at[idx], sem))
        pend.append(pltpu.async_copy(zeros_f, wgtg_hbm.at[idx], sem))
        pend.append(pltpu.async_copy(zeros_i, typeg_hbm.at[idx], sem))
        pend.append(pltpu.async_copy(paddst.at[c], dstg_hbm.at[idx], sem))
    for cp in pend:
        cp.wait()


@functools.partial(
    pl.kernel, mesh=_sc_mesh,
    compiler_params=pltpu.CompilerParams(needs_layout_passes=False),
    out_type=(jax.ShapeDtypeStruct((NPAD, H), jnp.float32),
              jax.ShapeDtypeStruct((NPAD, H), jnp.float32),
              jax.ShapeDtypeStruct((NPAD, H), jnp.float32),
              jax.ShapeDtypeStruct((NPAD, H), jnp.float32),
              jax.ShapeDtypeStruct((NPAD,), jnp.float32)),
    scratch_types=[
        pltpu.VMEM((136, H), jnp.float32),
        pltpu.VMEM((136, H), jnp.float32),
        pltpu.VMEM((136, H), jnp.float32),
        pltpu.VMEM((136, H), jnp.float32),
        pltpu.VMEM((136,), jnp.float32),
        pltpu.SMEM((136,), jnp.float32),
        pltpu.VMEM((2, KW, H), jnp.float32),
        pltpu.VMEM((MROWS + 8, 128), jnp.int32),
        pltpu.VMEM((MROWS + 8, 128), jnp.float32),
        pltpu.VMEM((MROWS + 8, 128), jnp.int32),
        pltpu.VMEM((MROWS + 8, 128), jnp.int32),
        pltpu.VMEM((4, H), jnp.float32),
        pltpu.VMEM((96,), jnp.int32),
        pltpu.SemaphoreType.DMA,
        pltpu.SemaphoreType.DMA,
    ],
)
def _sc_stats(b_hbm, t_hbm, srcg_hbm, wgtg_hbm, typeg_hbm, dstg_hbm, bs_hbm,
              s1_hbm, s2_hbm, mx_hbm, mn_hbm, deg_hbm,
              s1a, s2a, mxa, mna, degav, dega, rowsv,
              src2d, wgt2d, ty2d, dst2d, tv, startv, sem0, sem1):
    wid = _worker_id()
    pltpu.sync_copy(bs_hbm, startv.at[pl.ds(0, 88)])
    pltpu.sync_copy(t_hbm, tv)

    z16 = jnp.zeros((16,), jnp.float32)
    lo16 = jnp.full((16,), -1e30, jnp.float32)
    hi16 = jnp.full((16,), 1e30, jnp.float32)

    def do_round(rnd, _):
        b = wid + rnd * NW

        @pl.when(b < NBKT)
        def _process():
            base = b * BKTN
            sv = startv[pl.ds(b, 16)]
            start = sv[0]
            n_e = sv[1] - sv[0]
            nwin = lax.shift_right_logical(n_e, 7)
            r0 = lax.shift_right_logical(start, 7)
            roff = lax.bitwise_and(r0, 7)
            r0a = pl.multiple_of(r0 - roff, 8)

            pltpu.sync_copy(srcg_hbm.at[pl.ds(r0a, MROWS + 8)], src2d)
            pltpu.sync_copy(wgtg_hbm.at[pl.ds(r0a, MROWS + 8)], wgt2d)
            pltpu.sync_copy(typeg_hbm.at[pl.ds(r0a, MROWS + 8)], ty2d)
            pltpu.sync_copy(dstg_hbm.at[pl.ds(r0a, MROWS + 8)], dst2d)

            def zero(d, _):
                for j in range(8):
                    sl = pl.ds(j * 16, 16)
                    s1a[d, sl] = z16
                    s2a[d, sl] = z16
                    mxa[d, sl] = lo16
                    mna[d, sl] = hi16
                dega[d] = jnp.float32(0.0)
                return 0
            lax.fori_loop(0, 136, zero, 0)

            @pl.when(nwin > 0)
            def _prime():
                pltpu.async_copy(b_hbm.at[src2d.at[roff]], rowsv.at[0], sem0)

            def window(w, _):
                par = lax.bitwise_and(w, 1)
                nxt = w + 1

                @pl.when((nxt < nwin) & (lax.bitwise_and(nxt, 1) == 0))
                def _fire0():
                    pltpu.async_copy(b_hbm.at[src2d.at[roff + nxt]],
                                     rowsv.at[0], sem0)

                @pl.when((nxt < nwin) & (lax.bitwise_and(nxt, 1) == 1))
                def _fire1():
                    pltpu.async_copy(b_hbm.at[src2d.at[roff + nxt]],
                                     rowsv.at[1], sem1)

                @pl.when(par == 0)
                def _wait0():
                    pltpu.make_async_copy(b_hbm.at[pl.ds(0, KW)],
                                          rowsv.at[0], sem0).wait()

                @pl.when(par == 1)
                def _wait1():
                    pltpu.make_async_copy(b_hbm.at[pl.ds(0, KW)],
                                          rowsv.at[1], sem1).wait()

                def group(gq, _):
                    sl16 = pl.ds(gq * 16, 16)
                    dvec = dst2d[roff + w, sl16] - base
                    wvec = wgt2d[roff + w, sl16]
                    tyvec = ty2d[roff + w, sl16]
                    for k in range(16):
                        e = gq * 16 + k
                        d = dvec[k]
                        ty = tyvec[k]
                        wbc = wvec.at[jnp.full((16,), k, jnp.int32)].get(
                            mode="promise_in_bounds")
                        dega[d] = dega[d] + 1.0
                        for j in range(8):
                            sl = pl.ds(j * 16, 16)
                            r = rowsv[par, e, sl] + wbc * tv[ty, sl]
                            plsc.addupdate(s1a.at[d, sl], r)
                            plsc.addupdate(s2a.at[d, sl], r * r)
                            mxa[d, sl] = jnp.maximum(mxa[d, sl], r)
                            mna[d, sl] = jnp.minimum(mna[d, sl], r)
                    return 0
                lax.fori_loop(0, 8, group, 0)
                return 0
            lax.fori_loop(0, nwin, window, 0)

            def deg_tovmem(d, _):
                _sstore(degav, d, dega[d], jnp.float32)
                return 0
            lax.fori_loop(0, BKTN, deg_tovmem, 0)

            dsl = pl.ds(0, BKTN)
            osl = pl.ds(pl.multiple_of(base, BKTN), BKTN)
            pltpu.sync_copy(s1a.at[dsl], s1_hbm.at[osl])
            pltpu.sync_copy(s2a.at[dsl], s2_hbm.at[osl])
            pltpu.sync_copy(mxa.at[dsl], mx_hbm.at[osl])
            pltpu.sync_copy(mna.at[dsl], mn_hbm.at[osl])
            pltpu.sync_copy(degav.at[dsl], deg_hbm.at[osl])
        return 0
    lax.fori_loop(0, 3, do_round, 0)


def _mm_bias_body(x_ref, w_ref, b_ref, o_ref):
    o_ref[...] = x_ref[...] @ w_ref[...] + b_ref[...]


def _mm_bias(x, w, b, bm):
    m = x.shape[0]
    return pl.pallas_call(
        _mm_bias_body,
        grid=(m // bm,),
        in_specs=[pl.BlockSpec((bm, x.shape[1]), lambda i: (i, 0)),
                  pl.BlockSpec(w.shape, lambda i: (0, 0)),
                  pl.BlockSpec((1, w.shape[1]), lambda i: (0, 0))],
        out_specs=pl.BlockSpec((bm, w.shape[1]), lambda i: (i, 0)),
        out_shape=jax.ShapeDtypeStruct((m, w.shape[1]), jnp.float32),
    )(x, w, b[None, :])


def _ab_body(x_ref, wpre_ref, wedge_ref, eemb_ref, bedge_ref, bpre_ref,
             a_ref, b_ref, t_ref):
    w1 = wpre_ref[0:H, :]
    w2 = wpre_ref[H:2 * H, :]
    w3 = wpre_ref[2 * H:3 * H, :]
    x = x_ref[...]
    a_ref[...] = x @ w1
    c0 = bedge_ref[...] @ w3 + bpre_ref[...]
    b_ref[...] = x @ w2 + c0
    t_ref[...] = (eemb_ref[...] @ wedge_ref[...]) @ w3


def _ab_stage(entity, lp, eemb):
    bm = 1000
    return pl.pallas_call(
        _ab_body,
        grid=(N_NODES // bm,),
        in_specs=[pl.BlockSpec((bm, H), lambda i: (i, 0)),
                  pl.BlockSpec((3 * H, H), lambda i: (0, 0)),
                  pl.BlockSpec((16, H), lambda i: (0, 0)),
                  pl.BlockSpec((4, 16), lambda i: (0, 0)),
                  pl.BlockSpec((1, H), lambda i: (0, 0)),
                  pl.BlockSpec((1, H), lambda i: (0, 0))],
        out_specs=(pl.BlockSpec((bm, H), lambda i: (i, 0)),
                   pl.BlockSpec((bm, H), lambda i: (i, 0)),
                   pl.BlockSpec((4, H), lambda i: (0, 0))),
        out_shape=(jax.ShapeDtypeStruct((N_NODES, H), jnp.float32),
                   jax.ShapeDtypeStruct((N_NODES, H), jnp.float32),
                   jax.ShapeDtypeStruct((4, H), jnp.float32)),
    )(entity, lp['W_pre'], lp['W_edge'], eemb, lp['b_edge'][None, :],
      lp['b_pre'][None, :])


def _post_body(x_ref, a_ref, s1_ref, s2_ref, mx_ref, mn_ref, deg_ref,
               adl_ref, wpost_ref, bpost_ref, wlin_ref, blin_ref, o_ref):
    deg = deg_ref[...]
    degc = jnp.maximum(deg, 1.0)
    idegc = 1.0 / degc
    has = deg > 0
    a = a_ref[...]
    s1n = s1_ref[...] * idegc
    mean = jnp.where(has, a + s1n, 0.0)
    std = jnp.sqrt(jax.nn.relu(s2_ref[...] * idegc - s1n * s1n) + 1e-5)
    mx = jnp.where(has, a + mx_ref[...], 0.0)
    mn = jnp.where(has, a + mn_ref[...], 0.0)
    agg = jnp.concatenate([mean, mn, mx, std], axis=-1)
    adl = adl_ref[0, 0]
    lg = jnp.log(degc + 1.0)
    amp = lg * (1.0 / adl)
    att = adl / lg
    cat = jnp.concatenate([x_ref[...], agg, agg * amp, agg * att], axis=-1)
    out = cat @ wpost_ref[...] + bpost_ref[...]
    out = out @ wlin_ref[...] + blin_ref[...]
    o_ref[...] = _leaky(out)


def _post_stage(entity, a, s1, s2, mx, mn, deg, lp):
    bm = 1000
    nb = N_NODES // bm
    row = lambda i: (i, 0)
    fixed = lambda i: (0, 0)
    return pl.pallas_call(
        _post_body,
        grid=(nb,),
        in_specs=[pl.BlockSpec((bm, H), row),
                  pl.BlockSpec((bm, H), row),
                  pl.BlockSpec((bm, H), row),
                  pl.BlockSpec((bm, H), row),
                  pl.BlockSpec((bm, H), row),
                  pl.BlockSpec((bm, H), row),
                  pl.BlockSpec((bm, 1), row),
                  pl.BlockSpec((1, 1), fixed),
                  pl.BlockSpec((13 * H, H), fixed),
                  pl.BlockSpec((1, H), fixed),
                  pl.BlockSpec((H, H), fixed),
                  pl.BlockSpec((1, H), fixed)],
        out_specs=pl.BlockSpec((bm, H), row),
        out_shape=jax.ShapeDtypeStruct((N_NODES, H), jnp.float32),
    )(entity, a, s1, s2, mx, mn, deg[:, None],
      jnp.reshape(lp['avg_deg_log'], (1, 1)), lp['W_post'],
      lp['b_post'][None, :], lp['W_lin'], lp['b_lin'][None, :])


def _heads_body(comp_ref, prot_ref, wfc_ref, bfc_ref, wfp_ref, bfp_ref,
                w0_ref, w1_ref, u_ref, v_ref):
    fc = _leaky(comp_ref[...] @ wfc_ref[...] + bfc_ref[...])
    fc = fc / jnp.clip(jnp.sqrt(jnp.sum(fc * fc, axis=1, keepdims=True)), 1e-12)
    u_ref[...] = fc @ w0_ref[...]
    fp = _leaky(prot_ref[...] @ wfp_ref[...] + bfp_ref[...])
    fp = fp / jnp.clip(jnp.sqrt(jnp.sum(fp * fp, axis=1, keepdims=True)), 1e-12)
    v_ref[...] = fp @ w1_ref[...]


def _final_stage(comp, prot, params):
    return pl.pallas_call(
        _heads_body,
        out_shape=(jax.ShapeDtypeStruct((N_COMPOUND, 64), jnp.float32),
                   jax.ShapeDtypeStruct((N_PROTEIN, 64), jnp.float32)),
    )(comp, prot, params['Wfc'], params['bfc'][None, :], params['Wfp'],
      params['bfp'][None, :], params['W0p'], params['W1p'])


def _recon_body(u_ref, v_ref, cpi_ref, mask_ref, recon_ref, loss_ref):
    i = pl.program_id(0)
    j = pl.program_id(1)
    recon = u_ref[...] @ v_ref[...].T
    recon_ref[...] = recon
    tmp = mask_ref[...] * (recon - cpi_ref[...])
    part = jnp.sum(tmp * tmp)

    @pl.when((i == 0) & (j == 0))
    def _init():
        loss_ref[0, 0] = 0.0

    loss_ref[0, 0] += part


def _recon_loss(u, v, CPI, CPI_mask):
    bm, bn = 280, 3000
    grid = (N_COMPOUND // bm, N_PROTEIN // bn)
    recon, loss = pl.pallas_call(
        _recon_body,
        grid=grid,
        in_specs=[
            pl.BlockSpec((bm, 64), lambda i, j: (i, 0)),
            pl.BlockSpec((bn, 64), lambda i, j: (j, 0)),
            pl.BlockSpec((bm, bn), lambda i, j: (i, j)),
            pl.BlockSpec((bm, bn), lambda i, j: (i, j)),
        ],
        out_specs=(pl.BlockSpec((bm, bn), lambda i, j: (i, j)),
                   pl.BlockSpec((1, 1), lambda i, j: (0, 0),
                                memory_space=pltpu.SMEM)),
        out_shape=(jax.ShapeDtypeStruct((N_COMPOUND, N_PROTEIN), jnp.float32),
                   jax.ShapeDtypeStruct((1, 1), jnp.float32)),
    )(u, v, CPI, CPI_mask)
    return loss[0, 0], recon


def kernel(compound_embedding, protein_embedding, edge_index, edge_weight,
           edge_type, CPI, CPI_mask, params):
    src = edge_index[0]
    dst = edge_index[1]
    ew = edge_weight[:, 0]
    etype = edge_type.astype(jnp.int32)

    compound = _mm_bias(compound_embedding, params['Wc'], params['bc'], 1000)
    protein = _mm_bias(protein_embedding, params['Wp'], params['bp'], 1000)
    entity = jnp.concatenate([compound, protein], axis=0)
    entity0 = entity

    cnt = _sc_histogram(dst)
    src_g, wgt_g, type_g, dst_g, bkt_start = _sc_permute(src, ew, etype,
                                                         dst, cnt)
    src_g = src_g.reshape(NR, 128)
    wgt_g = wgt_g.reshape(NR, 128)
    type_g = type_g.reshape(NR, 128)
    dst_g = dst_g.reshape(NR, 128)

    for lp in params['layers']:
        a, b_eff, t_tab = _ab_stage(entity, lp, params['edge_emb'])
        s1, s2, mx, mn, deg = _sc_stats(b_eff, t_tab, src_g, wgt_g, type_g,
                                        dst_g, bkt_start)
        entity = _post_stage(entity, a, s1[:N_NODES], s2[:N_NODES],
                             mx[:N_NODES], mn[:N_NODES], deg[:N_NODES], lp)

    entity = jnp.concatenate([entity, entity0], axis=1)
    comp = entity[:N_COMPOUND]
    prot = entity[N_COMPOUND:N_COMPOUND + N_PROTEIN]
    u, v = _final_stage(comp, prot, params)
    return _recon_loss(u, v, CPI, CPI_mask)


# speedup vs baseline: 1.2473x; 1.0679x over previous
import functools

import jax
import jax.numpy as jnp
from jax import lax
from jax.experimental import pallas as pl
from jax.experimental.pallas import tpu as pltpu
from jax.experimental.pallas import tpu_sc as plsc

N_COMPOUND = 7000
N_PROTEIN = 3000
N_NODES = 10000
N_EDGES = 320000
H = 128
NEG = 0.01

NC, NS, L = 2, 16, 16
NW = NC * NS
EPW = N_EDGES // NW
NBKT = 79
BKTN = 128
NPAD = NBKT * BKTN
E_PAD = N_EDGES + NBKT * 128
E_ALL = E_PAD + 6144
NR = E_ALL // 128
TAIL = E_PAD
KW = 128
MROWS = 40
NCH = EPW // 128 + 1


def _leaky(x):
    return jnp.where(x >= 0, x, NEG * x)


def _worker_id():
    return lax.axis_index("s") * NC + lax.axis_index("c")


_LANE0 = None


def _lane0():
    return lax.broadcasted_iota(jnp.int32, (16,), 0) == 0


def _sstore(ref1d, index, value, dtype):
    plsc.store_scatter(ref1d, [jnp.full((16,), index, jnp.int32)],
                       jnp.full((16,), value, dtype), mask=_lane0())


def _sstore2(ref2d, row, col, value, dtype):
    plsc.store_scatter(ref2d,
                       [jnp.full((16,), row, jnp.int32),
                        jnp.full((16,), col, jnp.int32)],
                       jnp.full((16,), value, dtype), mask=_lane0())


_sc_mesh = plsc.VectorSubcoreMesh(core_axis_name="c", subcore_axis_name="s")


@functools.partial(
    pl.kernel, mesh=_sc_mesh,
    compiler_params=pltpu.CompilerParams(needs_layout_passes=False),
    out_type=jax.ShapeDtypeStruct((NW, 80), jnp.int32),
    scratch_types=[pltpu.VMEM((EPW,), jnp.int32), pltpu.VMEM((80,), jnp.int32),
                   pltpu.SMEM((80,), jnp.int32)],
)
def _sc_histogram(dst_hbm, cnt_hbm, dstv, histv, hists):
    wid = _worker_id()
    pltpu.sync_copy(dst_hbm.at[pl.ds(pl.multiple_of(wid * EPW, 8), EPW)], dstv)

    def zero(i, _):
        hists[i] = 0
        return 0
    lax.fori_loop(0, 80, zero, 0)

    def count(g, _):
        bvec = lax.shift_right_logical(dstv[pl.ds(g * 16, 16)], 7)
        for k in range(16):
            b = bvec[k]
            hists[b] = hists[b] + 1
        return 0
    lax.fori_loop(0, EPW // 16, count, 0)

    def tovmem(i, _):
        _sstore(histv, i, hists[i], jnp.int32)
        return 0
    lax.fori_loop(0, 80, tovmem, 0)
    pltpu.sync_copy(histv, cnt_hbm.at[wid])


@functools.partial(
    pl.kernel, mesh=_sc_mesh,
    compiler_params=pltpu.CompilerParams(needs_layout_passes=False),
    out_type=(jax.ShapeDtypeStruct((E_ALL,), jnp.int32),
              jax.ShapeDtypeStruct((E_ALL,), jnp.float32),
              jax.ShapeDtypeStruct((88,), jnp.int32)),
    scratch_types=[
        pltpu.VMEM((NCH * 128,), jnp.int32),
        pltpu.VMEM((NCH * 128,), jnp.float32),
        pltpu.VMEM((NCH * 128,), jnp.int32),
        pltpu.VMEM((NCH * 128,), jnp.int32),
        pltpu.VMEM((NW, 80), jnp.int32),
        pltpu.VMEM((88,), jnp.int32),
        pltpu.SMEM((88,), jnp.int32),
        pltpu.SMEM((80,), jnp.int32),
        pltpu.SMEM((80,), jnp.int32),
        pltpu.VMEM((NCH, 128), jnp.int32),
        pltpu.VMEM((3, 128), jnp.int32),
        pltpu.VMEM((NCH * 128,), jnp.int32),
        pltpu.VMEM((128,), jnp.int32),
        pltpu.VMEM((128,), jnp.float32),
        pltpu.SemaphoreType.DMA,
    ],
)
def _sc_permute(src_hbm, wgt_hbm, type_hbm, dst_hbm, cnt_hbm,
                pkg_hbm, wgtg_hbm, bs_hbm,
                srcv, wgtv, typev, dstv, cntv, startv, starts, curs, tots,
                pos2d, padpos, packedv, padpk, zeros_f, sem):
    wid = _worker_id()
    pltpu.sync_copy(cnt_hbm, cntv)
    pltpu.sync_copy(src_hbm.at[pl.ds(pl.multiple_of(wid * EPW, 8), EPW)], srcv.at[pl.ds(0, EPW)])
    pltpu.sync_copy(wgt_hbm.at[pl.ds(pl.multiple_of(wid * EPW, 8), EPW)], wgtv.at[pl.ds(0, EPW)])
    pltpu.sync_copy(type_hbm.at[pl.ds(pl.multiple_of(wid * EPW, 8), EPW)], typev.at[pl.ds(0, EPW)])
    pltpu.sync_copy(dst_hbm.at[pl.ds(pl.multiple_of(wid * EPW, 8), EPW)], dstv.at[pl.ds(0, EPW)])

    zero16 = jnp.zeros((16,), jnp.int32)
    tot_chunks = []
    mine_chunks = []
    for j in range(5):
        sl = pl.ds(j * 16, 16)

        def addall(t, acc, sl=sl):
            return acc + cntv[t, sl]
        tot_chunks.append(lax.fori_loop(0, NW, addall, zero16))
        mine_chunks.append(lax.fori_loop(0, wid, addall, zero16))

    acc = jnp.int32(0)
    for b in range(NBKT):
        tot_b = tot_chunks[b // 16][b % 16]
        mine_b = mine_chunks[b // 16][b % 16]
        starts[b] = acc
        curs[b] = acc + mine_b
        tots[b] = tot_b
        acc = acc + lax.bitwise_and(tot_b + 127, -128)
    for b in range(NBKT, 88):
        starts[b] = acc

    def starts_tovmem(i, _):
        _sstore(startv, i, starts[i], jnp.int32)
        return 0
    lax.fori_loop(0, 88, starts_tovmem, 0)

    @pl.when(wid == 0)
    def _():
        pltpu.sync_copy(startv, bs_hbm)

    z16f = jnp.zeros((16,), jnp.float32)
    pk16 = jnp.full((16,), 128 << 16, jnp.int32)
    iota16c = lax.broadcasted_iota(jnp.int32, (16,), 0)
    for j in range(8):
        sl = pl.ds(j * 16, 16)
        padpk[sl] = pk16
        zeros_f[sl] = z16f
    for c in range(3):
        for j in range(8):
            sl = pl.ds(j * 16, 16)
            padpos[c, sl] = jnp.full((16,), TAIL + 512, jnp.int32) + \
                iota16c + (c * 128 + j * 16)

    def mkpack(g, _):
        sl = pl.ds(g * 16, 16)
        packedv[sl] = jnp.bitwise_or(
            jnp.bitwise_or(srcv[sl], lax.shift_left(typev[sl], 14)),
            lax.shift_left(lax.bitwise_and(dstv[sl], 127), 16))
        return 0
    lax.fori_loop(0, EPW // 16, mkpack, 0)
    for j in range(1, 8):
        sl = pl.ds((NCH - 1) * 128 + j * 16, 16)
        packedv[sl] = pk16

    iota16 = lax.broadcasted_iota(jnp.int32, (16,), 0)
    for j in range(1, 8):
        pos2d[NCH - 1, pl.ds(j * 16, 16)] = TAIL + iota16 + j * 16

    def rank(g, _):
        gb = g * 16
        bvec = lax.shift_right_logical(dstv[pl.ds(gb, 16)], 7)
        row = lax.shift_right_logical(gb, 7)
        col0 = lax.bitwise_and(gb, 127)
        acc = zero16
        for k in range(16):
            b = bvec[k]
            p = curs[b]
            curs[b] = p + 1
            acc = jnp.where(iota16 == k, p, acc)
        pos2d[row, pl.ds(col0, 16)] = acc
        return 0
    lax.fori_loop(0, EPW // 16, rank, 0)

    def fill_bucket(rnd, slot):
        b = wid + rnd * NW
        b_c = jnp.minimum(b, NBKT - 1)
        gapstart = starts[b_c] + tots[b_c]
        gap = jnp.where(b < NBKT, starts[b_c + 1] - gapstart, 0)

        def put(g, slot):
            _sstore2(padpos, lax.shift_right_logical(slot, 7),
                     lax.bitwise_and(slot, 127), gapstart + g, jnp.int32)
            return slot + 1
        return lax.fori_loop(0, gap, put, slot)
    lax.fori_loop(0, 3, fill_bucket, 0)

    pend = []
    for j in range(NCH):
        sl = pl.ds(j * 128, 128)
        idx = pos2d.at[j]
        pend.append(pltpu.async_copy(packedv.at[sl], pkg_hbm.at[idx], sem))
        pend.append(pltpu.async_copy(wgtv.at[sl], wgtg_hbm.at[idx], sem))
        if len(pend) >= 64:
            for cp in pend:
                cp.wait()
            pend = []
    for c in range(3):
        idx = padpos.at[c]
        pend.append(pltpu.async_copy(padpk, pkg_hbm.at[idx], sem))
        pend.append(pltpu.async_copy(zeros_f, wgtg_hbm.at[idx], sem))
    for cp in pend:
        cp.wait()


@functools.partial(
    pl.kernel, mesh=_sc_mesh,
    compiler_params=pltpu.CompilerParams(needs_layout_passes=False),
    out_type=(jax.ShapeDtypeStruct((NPAD, H), jnp.float32),
              jax.ShapeDtypeStruct((NPAD, H), jnp.float32),
              jax.ShapeDtypeStruct((NPAD, H), jnp.float32),
              jax.ShapeDtypeStruct((NPAD, H), jnp.float32),
              jax.ShapeDtypeStruct((NPAD,), jnp.float32)),
    scratch_types=[
        pltpu.VMEM((136, H), jnp.float32),
        pltpu.VMEM((136, H), jnp.float32),
        pltpu.VMEM((136, H), jnp.float32),
        pltpu.VMEM((136, H), jnp.float32),
        pltpu.VMEM((136,), jnp.float32),
        pltpu.SMEM((136,), jnp.float32),
        pltpu.VMEM((2, KW, H), jnp.float32),
        pltpu.VMEM((MROWS + 8, 128), jnp.int32),
        pltpu.VMEM((MROWS + 8, 128), jnp.float32),
        pltpu.VMEM((2, 128), jnp.int32),
        pltpu.VMEM((4, H), jnp.float32),
        pltpu.VMEM((96,), jnp.int32),
        pltpu.SemaphoreType.DMA,
        pltpu.SemaphoreType.DMA,
    ],
)
def _sc_stats(b_hbm, t_hbm, pkg_hbm, wgtg_hbm, bs_hbm,
              s1_hbm, s2_hbm, mx_hbm, mn_hbm, deg_hbm,
              s1a, s2a, mxa, mna, degav, dega, rowsv,
              pk2d, wg2d, srcidx, tv, startv, sem0, sem1):
    wid = _worker_id()
    pltpu.sync_copy(bs_hbm, startv.at[pl.ds(0, 88)])
    pltpu.sync_copy(t_hbm, tv)

    z16 = jnp.zeros((16,), jnp.float32)
    lo16 = jnp.full((16,), -1e30, jnp.float32)
    hi16 = jnp.full((16,), 1e30, jnp.float32)

    def do_round(rnd, _):
        b = wid + rnd * NW

        @pl.when(b < NBKT)
        def _process():
            base = b * BKTN
            sv = startv[pl.ds(b, 16)]
            start = sv[0]
            n_e = sv[1] - sv[0]
            nwin = lax.shift_right_logical(n_e, 7)
            r0 = lax.shift_right_logical(start, 7)
            roff = lax.bitwise_and(r0, 7)
            r0a = pl.multiple_of(r0 - roff, 8)

            pltpu.sync_copy(pkg_hbm.at[pl.ds(r0a, MROWS + 8)], pk2d)
            pltpu.sync_copy(wgtg_hbm.at[pl.ds(r0a, MROWS + 8)], wg2d)

            def zero(d, _):
                for j in range(8):
                    sl = pl.ds(j * 16, 16)
                    s1a[d, sl] = z16
                    s2a[d, sl] = z16
                    mxa[d, sl] = lo16
                    mna[d, sl] = hi16
                dega[d] = jnp.float32(0.0)
                return 0
            lax.fori_loop(0, 136, zero, 0)

            iota16b = lax.broadcasted_iota(jnp.int32, (16,), 0)

            def build_srcidx(w):
                par = lax.bitwise_and(w, 1)
                for gq in range(8):
                    sl = pl.ds(gq * 16, 16)
                    srcidx[par, sl] = jnp.bitwise_and(pk2d[roff + w, sl],
                                                      16383)

            @pl.when(nwin > 0)
            def _prime():
                build_srcidx(0)
                pltpu.async_copy(b_hbm.at[srcidx.at[0]], rowsv.at[0], sem0)

            def window(w, _):
                par = lax.bitwise_and(w, 1)
                nxt = w + 1

                @pl.when((nxt < nwin) & (lax.bitwise_and(nxt, 1) == 0))
                def _fire0():
                    build_srcidx(nxt)
                    pltpu.async_copy(b_hbm.at[srcidx.at[0]],
                                     rowsv.at[0], sem0)

                @pl.when((nxt < nwin) & (lax.bitwise_and(nxt, 1) == 1))
                def _fire1():
                    build_srcidx(nxt)
                    pltpu.async_copy(b_hbm.at[srcidx.at[1]],
                                     rowsv.at[1], sem1)

                @pl.when(par == 0)
                def _wait0():
                    pltpu.make_async_copy(b_hbm.at[pl.ds(0, KW)],
                                          rowsv.at[0], sem0).wait()

                @pl.when(par == 1)
                def _wait1():
                    pltpu.make_async_copy(b_hbm.at[pl.ds(0, KW)],
                                          rowsv.at[1], sem1).wait()

                def group(gq, _):
                    sl16 = pl.ds(gq * 16, 16)
                    pvec = pk2d[roff + w, sl16]
                    dvec = lax.shift_right_logical(pvec, 16)
                    wvec = wg2d[roff + w, sl16]
                    tyvec = jnp.bitwise_and(lax.shift_right_logical(pvec, 14),
                                            3)
                    for k in range(16):
                        e = gq * 16 + k
                        d = dvec[k]
                        ty = tyvec[k]
                        wbc = wvec.at[jnp.full((16,), k, jnp.int32)].get(
                            mode="promise_in_bounds")
                        dega[d] = dega[d] + 1.0
                        for j in range(8):
                            sl = pl.ds(j * 16, 16)
                            r = rowsv[par, e, sl] + wbc * tv[ty, sl]
                            plsc.addupdate(s1a.at[d, sl], r)
                            plsc.addupdate(s2a.at[d, sl], r * r)
                            mxa[d, sl] = jnp.maximum(mxa[d, sl], r)
                            mna[d, sl] = jnp.minimum(mna[d, sl], r)
                    return 0
                lax.fori_loop(0, 8, group, 0)
                return 0
            lax.fori_loop(0, nwin, window, 0)

            def deg_tovmem(d, _):
                _sstore(degav, d, dega[d], jnp.float32)
                return 0
            lax.fori_loop(0, BKTN, deg_tovmem, 0)

            dsl = pl.ds(0, BKTN)
            osl = pl.ds(pl.multiple_of(base, BKTN), BKTN)
            pltpu.sync_copy(s1a.at[dsl], s1_hbm.at[osl])
            pltpu.sync_copy(s2a.at[dsl], s2_hbm.at[osl])
            pltpu.sync_copy(mxa.at[dsl], mx_hbm.at[osl])
            pltpu.sync_copy(mna.at[dsl], mn_hbm.at[osl])
            pltpu.sync_copy(degav.at[dsl], deg_hbm.at[osl])
        return 0
    lax.fori_loop(0, 3, do_round, 0)


def _mm_bias_body(x_ref, w_ref, b_ref, o_ref):
    o_ref[...] = x_ref[...] @ w_ref[...] + b_ref[...]


def _mm_bias(x, w, b, bm):
    m = x.shape[0]
    return pl.pallas_call(
        _mm_bias_body,
        grid=(m // bm,),
        in_specs=[pl.BlockSpec((bm, x.shape[1]), lambda i: (i, 0)),
                  pl.BlockSpec(w.shape, lambda i: (0, 0)),
                  pl.BlockSpec((1, w.shape[1]), lambda i: (0, 0))],
        out_specs=pl.BlockSpec((bm, w.shape[1]), lambda i: (i, 0)),
        out_shape=jax.ShapeDtypeStruct((m, w.shape[1]), jnp.float32),
    )(x, w, b[None, :])


def _ab_body(x_ref, wpre_ref, wedge_ref, eemb_ref, bedge_ref, bpre_ref,
             a_ref, b_ref, t_ref):
    w1 = wpre_ref[0:H, :]
    w2 = wpre_ref[H:2 * H, :]
    w3 = wpre_ref[2 * H:3 * H, :]
    x = x_ref[...]
    a_ref[...] = x @ w1
    c0 = bedge_ref[...] @ w3 + bpre_ref[...]
    b_ref[...] = x @ w2 + c0
    t_ref[...] = (eemb_ref[...] @ wedge_ref[...]) @ w3


def _ab_stage(entity, lp, eemb):
    bm = 1000
    return pl.pallas_call(
        _ab_body,
        grid=(N_NODES // bm,),
        in_specs=[pl.BlockSpec((bm, H), lambda i: (i, 0)),
                  pl.BlockSpec((3 * H, H), lambda i: (0, 0)),
                  pl.BlockSpec((16, H), lambda i: (0, 0)),
                  pl.BlockSpec((4, 16), lambda i: (0, 0)),
                  pl.BlockSpec((1, H), lambda i: (0, 0)),
                  pl.BlockSpec((1, H), lambda i: (0, 0))],
        out_specs=(pl.BlockSpec((bm, H), lambda i: (i, 0)),
                   pl.BlockSpec((bm, H), lambda i: (i, 0)),
                   pl.BlockSpec((4, H), lambda i: (0, 0))),
        out_shape=(jax.ShapeDtypeStruct((N_NODES, H), jnp.float32),
                   jax.ShapeDtypeStruct((N_NODES, H), jnp.float32),
                   jax.ShapeDtypeStruct((4, H), jnp.float32)),
    )(entity, lp['W_pre'], lp['W_edge'], eemb, lp['b_edge'][None, :],
      lp['b_pre'][None, :])


def _post_body(x_ref, a_ref, s1_ref, s2_ref, mx_ref, mn_ref, deg_ref,
               adl_ref, wpost_ref, bpost_ref, wlin_ref, blin_ref, o_ref):
    deg = deg_ref[...]
    degc = jnp.maximum(deg, 1.0)
    idegc = 1.0 / degc
    has = deg > 0
    a = a_ref[...]
    s1n = s1_ref[...] * idegc
    mean = jnp.where(has, a + s1n, 0.0)
    std = jnp.sqrt(jax.nn.relu(s2_ref[...] * idegc - s1n * s1n) + 1e-5)
    mx = jnp.where(has, a + mx_ref[...], 0.0)
    mn = jnp.where(has, a + mn_ref[...], 0.0)
    agg = jnp.concatenate([mean, mn, mx, std], axis=-1)
    adl = adl_ref[0, 0]
    lg = jnp.log(degc + 1.0)
    amp = lg * (1.0 / adl)
    att = adl / lg
    cat = jnp.concatenate([x_ref[...], agg, agg * amp, agg * att], axis=-1)
    out = cat @ wpost_ref[...] + bpost_ref[...]
    out = out @ wlin_ref[...] + blin_ref[...]
    o_ref[...] = _leaky(out)


def _post_stage(entity, a, s1, s2, mx, mn, deg, lp):
    bm = 1000
    nb = N_NODES // bm
    row = lambda i: (i, 0)
    fixed = lambda i: (0, 0)
    return pl.pallas_call(
        _post_body,
        grid=(nb,),
        in_specs=[pl.BlockSpec((bm, H), row),
                  pl.BlockSpec((bm, H), row),
                  pl.BlockSpec((bm, H), row),
                  pl.BlockSpec((bm, H), row),
                  pl.BlockSpec((bm, H), row),
                  pl.BlockSpec((bm, H), row),
                  pl.BlockSpec((bm, 1), row),
                  pl.BlockSpec((1, 1), fixed),
                  pl.BlockSpec((13 * H, H), fixed),
                  pl.BlockSpec((1, H), fixed),
                  pl.BlockSpec((H, H), fixed),
                  pl.BlockSpec((1, H), fixed)],
        out_specs=pl.BlockSpec((bm, H), row),
        out_shape=jax.ShapeDtypeStruct((N_NODES, H), jnp.float32),
    )(entity, a, s1, s2, mx, mn, deg[:, None],
      jnp.reshape(lp['avg_deg_log'], (1, 1)), lp['W_post'],
      lp['b_post'][None, :], lp['W_lin'], lp['b_lin'][None, :])


def _heads_body(comp_ref, prot_ref, wfc_ref, bfc_ref, wfp_ref, bfp_ref,
                w0_ref, w1_ref, u_ref, v_ref):
    fc = _leaky(comp_ref[...] @ wfc_ref[...] + bfc_ref[...])
    fc = fc / jnp.clip(jnp.sqrt(jnp.sum(fc * fc, axis=1, keepdims=True)), 1e-12)
    u_ref[...] = fc @ w0_ref[...]
    fp = _leaky(prot_ref[...] @ wfp_ref[...] + bfp_ref[...])
    fp = fp / jnp.clip(jnp.sqrt(jnp.sum(fp * fp, axis=1, keepdims=True)), 1e-12)
    v_ref[...] = fp @ w1_ref[...]


def _final_stage(comp, prot, params):
    return pl.pallas_call(
        _heads_body,
        out_shape=(jax.ShapeDtypeStruct((N_COMPOUND, 64), jnp.float32),
                   jax.ShapeDtypeStruct((N_PROTEIN, 64), jnp.float32)),
    )(comp, prot, params['Wfc'], params['bfc'][None, :], params['Wfp'],
      params['bfp'][None, :], params['W0p'], params['W1p'])


def _recon_body(u_ref, v_ref, cpi_ref, mask_ref, recon_ref, loss_ref):
    i = pl.program_id(0)
    j = pl.program_id(1)
    recon = u_ref[...] @ v_ref[...].T
    recon_ref[...] = recon
    tmp = mask_ref[...] * (recon - cpi_ref[...])
    part = jnp.sum(tmp * tmp)

    @pl.when((i == 0) & (j == 0))
    def _init():
        loss_ref[0, 0] = 0.0

    loss_ref[0, 0] += part


def _recon_loss(u, v, CPI, CPI_mask):
    bm, bn = 280, 3000
    grid = (N_COMPOUND // bm, N_PROTEIN // bn)
    recon, loss = pl.pallas_call(
        _recon_body,
        grid=grid,
        in_specs=[
            pl.BlockSpec((bm, 64), lambda i, j: (i, 0)),
            pl.BlockSpec((bn, 64), lambda i, j: (j, 0)),
            pl.BlockSpec((bm, bn), lambda i, j: (i, j)),
            pl.BlockSpec((bm, bn), lambda i, j: (i, j)),
        ],
        out_specs=(pl.BlockSpec((bm, bn), lambda i, j: (i, j)),
                   pl.BlockSpec((1, 1), lambda i, j: (0, 0),
                                memory_space=pltpu.SMEM)),
        out_shape=(jax.ShapeDtypeStruct((N_COMPOUND, N_PROTEIN), jnp.float32),
                   jax.ShapeDtypeStruct((1, 1), jnp.float32)),
    )(u, v, CPI, CPI_mask)
    return loss[0, 0], recon


def kernel(compound_embedding, protein_embedding, edge_index, edge_weight,
           edge_type, CPI, CPI_mask, params):
    src = edge_index[0]
    dst = edge_index[1]
    ew = edge_weight[:, 0]
    etype = edge_type.astype(jnp.int32)

    compound = _mm_bias(compound_embedding, params['Wc'], params['bc'], 1000)
    protein = _mm_bias(protein_embedding, params['Wp'], params['bp'], 1000)
    entity = jnp.concatenate([compound, protein], axis=0)
    entity0 = entity

    cnt = _sc_histogram(dst)
    packed_g, wgt_g, bkt_start = _sc_permute(src, ew, etype, dst, cnt)
    packed_g = packed_g.reshape(NR, 128)
    wgt_g = wgt_g.reshape(NR, 128)

    for lp in params['layers']:
        a, b_eff, t_tab = _ab_stage(entity, lp, params['edge_emb'])
        s1, s2, mx, mn, deg = _sc_stats(b_eff, t_tab, packed_g, wgt_g,
                                        bkt_start)
        entity = _post_stage(entity, a, s1[:N_NODES], s2[:N_NODES],
                             mx[:N_NODES], mn[:N_NODES], deg[:N_NODES], lp)

    entity = jnp.concatenate([entity, entity0], axis=1)
    comp = entity[:N_COMPOUND]
    prot = entity[N_COMPOUND:N_COMPOUND + N_PROTEIN]
    u, v = _final_stage(comp, prot, params)
    return _recon_loss(u, v, CPI, CPI_mask)


# speedup vs baseline: 1.2496x; 1.0018x over previous
import functools

import jax
import jax.numpy as jnp
from jax import lax
from jax.experimental import pallas as pl
from jax.experimental.pallas import tpu as pltpu
from jax.experimental.pallas import tpu_sc as plsc

N_COMPOUND = 7000
N_PROTEIN = 3000
N_NODES = 10000
N_EDGES = 320000
H = 128
NEG = 0.01

NC, NS, L = 2, 16, 16
NW = NC * NS
EPW = N_EDGES // NW
NBKT = 79
BKTN = 128
NPAD = NBKT * BKTN
E_PAD = N_EDGES + NBKT * 128
E_ALL = E_PAD + 6144
NR = E_ALL // 128
TAIL = E_PAD
KW = 128
MROWS = 40
NCH = EPW // 128 + 1


def _leaky(x):
    return jnp.where(x >= 0, x, NEG * x)


def _worker_id():
    return lax.axis_index("s") * NC + lax.axis_index("c")


def _lane0():
    return lax.broadcasted_iota(jnp.int32, (16,), 0) == 0


def _sstore(ref1d, index, value, dtype):
    plsc.store_scatter(ref1d, [jnp.full((16,), index, jnp.int32)],
                       jnp.full((16,), value, dtype), mask=_lane0())


def _sstore2(ref2d, row, col, value, dtype):
    plsc.store_scatter(ref2d,
                       [jnp.full((16,), row, jnp.int32),
                        jnp.full((16,), col, jnp.int32)],
                       jnp.full((16,), value, dtype), mask=_lane0())


_sc_mesh = plsc.VectorSubcoreMesh(core_axis_name="c", subcore_axis_name="s")


@functools.partial(
    pl.kernel, mesh=_sc_mesh,
    compiler_params=pltpu.CompilerParams(needs_layout_passes=False),
    out_type=jax.ShapeDtypeStruct((NW, 80), jnp.int32),
    scratch_types=[pltpu.VMEM((EPW,), jnp.int32), pltpu.VMEM((80,), jnp.int32),
                   pltpu.SMEM((80,), jnp.int32)],
)
def _sc_histogram(dst_hbm, cnt_hbm, dstv, histv, hists):
    wid = _worker_id()
    pltpu.sync_copy(dst_hbm.at[pl.ds(pl.multiple_of(wid * EPW, 8), EPW)], dstv)

    def zero(i, _):
        hists[i] = 0
        return 0
    lax.fori_loop(0, 80, zero, 0)

    def count(g, _):
        bvec = lax.shift_right_logical(dstv[pl.ds(g * 16, 16)], 7)
        for k in range(16):
            b = bvec[k]
            hists[b] = hists[b] + 1
        return 0
    lax.fori_loop(0, EPW // 16, count, 0)

    def tovmem(i, _):
        _sstore(histv, i, hists[i], jnp.int32)
        return 0
    lax.fori_loop(0, 80, tovmem, 0)
    pltpu.sync_copy(histv, cnt_hbm.at[wid])


@functools.partial(
    pl.kernel, mesh=_sc_mesh,
    compiler_params=pltpu.CompilerParams(needs_layout_passes=False),
    out_type=(jax.ShapeDtypeStruct((E_ALL,), jnp.int32),
              jax.ShapeDtypeStruct((E_ALL,), jnp.float32),
              jax.ShapeDtypeStruct((88,), jnp.int32)),
    scratch_types=[
        pltpu.VMEM((NCH * 128,), jnp.int32),
        pltpu.VMEM((NCH * 128,), jnp.float32),
        pltpu.VMEM((NCH * 128,), jnp.int32),
        pltpu.VMEM((NCH * 128,), jnp.int32),
        pltpu.VMEM((NW, 80), jnp.int32),
        pltpu.VMEM((88,), jnp.int32),
        pltpu.SMEM((88,), jnp.int32),
        pltpu.SMEM((80,), jnp.int32),
        pltpu.SMEM((80,), jnp.int32),
        pltpu.VMEM((NCH, 128), jnp.int32),
        pltpu.VMEM((3, 128), jnp.int32),
        pltpu.VMEM((NCH * 128,), jnp.int32),
        pltpu.VMEM((128,), jnp.int32),
        pltpu.VMEM((128,), jnp.float32),
        pltpu.SemaphoreType.DMA,
    ],
)
def _sc_permute(src_hbm, wgt_hbm, type_hbm, dst_hbm, cnt_hbm,
                pkg_hbm, wgtg_hbm, bs_hbm,
                srcv, wgtv, typev, dstv, cntv, startv, starts, curs, tots,
                pos2d, padpos, packedv, padpk, zeros_f, sem):
    wid = _worker_id()
    pltpu.sync_copy(cnt_hbm, cntv)
    pltpu.sync_copy(src_hbm.at[pl.ds(pl.multiple_of(wid * EPW, 8), EPW)], srcv.at[pl.ds(0, EPW)])
    pltpu.sync_copy(wgt_hbm.at[pl.ds(pl.multiple_of(wid * EPW, 8), EPW)], wgtv.at[pl.ds(0, EPW)])
    pltpu.sync_copy(type_hbm.at[pl.ds(pl.multiple_of(wid * EPW, 8), EPW)], typev.at[pl.ds(0, EPW)])
    pltpu.sync_copy(dst_hbm.at[pl.ds(pl.multiple_of(wid * EPW, 8), EPW)], dstv.at[pl.ds(0, EPW)])

    zero16 = jnp.zeros((16,), jnp.int32)
    tot_chunks = []
    mine_chunks = []
    for j in range(5):
        sl = pl.ds(j * 16, 16)

        def addall(t, acc, sl=sl):
            return acc + cntv[t, sl]
        tot_chunks.append(lax.fori_loop(0, NW, addall, zero16))
        mine_chunks.append(lax.fori_loop(0, wid, addall, zero16))

    acc = jnp.int32(0)
    for b in range(NBKT):
        tot_b = tot_chunks[b // 16][b % 16]
        mine_b = mine_chunks[b // 16][b % 16]
        starts[b] = acc
        curs[b] = acc + mine_b
        tots[b] = tot_b
        acc = acc + lax.bitwise_and(tot_b + 127, -128)
    for b in range(NBKT, 88):
        starts[b] = acc

    def starts_tovmem(i, _):
        _sstore(startv, i, starts[i], jnp.int32)
        return 0
    lax.fori_loop(0, 88, starts_tovmem, 0)

    @pl.when(wid == 0)
    def _():
        pltpu.sync_copy(startv, bs_hbm)

    z16f = jnp.zeros((16,), jnp.float32)
    pk16 = jnp.full((16,), 128 << 16, jnp.int32)
    iota16c = lax.broadcasted_iota(jnp.int32, (16,), 0)
    for j in range(8):
        sl = pl.ds(j * 16, 16)
        padpk[sl] = pk16
        zeros_f[sl] = z16f
    for c in range(3):
        for j in range(8):
            sl = pl.ds(j * 16, 16)
            padpos[c, sl] = jnp.full((16,), TAIL + 512, jnp.int32) + \
                iota16c + (c * 128 + j * 16)

    def mkpack(g, _):
        sl = pl.ds(g * 16, 16)
        packedv[sl] = jnp.bitwise_or(
            jnp.bitwise_or(srcv[sl], lax.shift_left(typev[sl], 14)),
            lax.shift_left(lax.bitwise_and(dstv[sl], 127), 16))
        return 0
    lax.fori_loop(0, EPW // 16, mkpack, 0)
    for j in range(1, 8):
        sl = pl.ds((NCH - 1) * 128 + j * 16, 16)
        packedv[sl] = pk16

    iota16 = lax.broadcasted_iota(jnp.int32, (16,), 0)
    for j in range(1, 8):
        pos2d[NCH - 1, pl.ds(j * 16, 16)] = TAIL + iota16 + j * 16

    def rank(g, _):
        gb = g * 16
        bvec = lax.shift_right_logical(dstv[pl.ds(gb, 16)], 7)
        row = lax.shift_right_logical(gb, 7)
        col0 = lax.bitwise_and(gb, 127)
        acc = zero16
        for k in range(16):
            b = bvec[k]
            p = curs[b]
            curs[b] = p + 1
            acc = jnp.where(iota16 == k, p, acc)
        pos2d[row, pl.ds(col0, 16)] = acc
        return 0
    lax.fori_loop(0, EPW // 16, rank, 0)

    def fill_bucket(rnd, slot):
        b = wid + rnd * NW
        b_c = jnp.minimum(b, NBKT - 1)
        gapstart = starts[b_c] + tots[b_c]
        gap = jnp.where(b < NBKT, starts[b_c + 1] - gapstart, 0)

        def put(g, slot):
            _sstore2(padpos, lax.shift_right_logical(slot, 7),
                     lax.bitwise_and(slot, 127), gapstart + g, jnp.int32)
            return slot + 1
        return lax.fori_loop(0, gap, put, slot)
    lax.fori_loop(0, 3, fill_bucket, 0)

    pend = []
    for j in range(NCH):
        sl = pl.ds(j * 128, 128)
        idx = pos2d.at[j]
        pend.append(pltpu.async_copy(packedv.at[sl], pkg_hbm.at[idx], sem))
        pend.append(pltpu.async_copy(wgtv.at[sl], wgtg_hbm.at[idx], sem))
        if len(pend) >= 64:
            for cp in pend:
                cp.wait()
            pend = []
    for c in range(3):
        idx = padpos.at[c]
        pend.append(pltpu.async_copy(padpk, pkg_hbm.at[idx], sem))
        pend.append(pltpu.async_copy(zeros_f, wgtg_hbm.at[idx], sem))
    for cp in pend:
        cp.wait()


@functools.partial(
    pl.kernel, mesh=_sc_mesh,
    compiler_params=pltpu.CompilerParams(needs_layout_passes=False),
    out_type=(jax.ShapeDtypeStruct((NPAD, H), jnp.float32),
              jax.ShapeDtypeStruct((NPAD, H), jnp.float32),
              jax.ShapeDtypeStruct((NPAD, H), jnp.float32),
              jax.ShapeDtypeStruct((NPAD, H), jnp.float32),
              jax.ShapeDtypeStruct((NPAD,), jnp.float32)),
    scratch_types=[
        pltpu.VMEM((136, H), jnp.float32),
        pltpu.VMEM((136, H), jnp.float32),
        pltpu.VMEM((136, H), jnp.float32),
        pltpu.VMEM((136, H), jnp.float32),
        pltpu.VMEM((136,), jnp.float32),
        pltpu.SMEM((136,), jnp.float32),
        pltpu.VMEM((2, KW, H), jnp.float32),
        pltpu.VMEM((MROWS + 8, 128), jnp.int32),
        pltpu.VMEM((MROWS + 8, 128), jnp.float32),
        pltpu.VMEM((2, 128), jnp.int32),
        pltpu.VMEM((4, H), jnp.float32),
        pltpu.VMEM((96,), jnp.int32),
        pltpu.SemaphoreType.DMA,
        pltpu.SemaphoreType.DMA,
    ],
)
def _sc_stats(b_hbm, t_hbm, pkg_hbm, wgtg_hbm, bs_hbm,
              s1_hbm, s2_hbm, mx_hbm, mn_hbm, deg_hbm,
              s1a, s2a, mxa, mna, degav, dega, rowsv,
              pk2d, wg2d, srcidx, tv, startv, sem0, sem1):
    wid = _worker_id()
    pltpu.sync_copy(bs_hbm, startv.at[pl.ds(0, 88)])
    pltpu.sync_copy(t_hbm, tv)

    z16 = jnp.zeros((16,), jnp.float32)
    lo16 = jnp.full((16,), -1e30, jnp.float32)
    hi16 = jnp.full((16,), 1e30, jnp.float32)

    def do_round(rnd, _):
        b = wid + rnd * NW

        @pl.when(b < NBKT)
        def _process():
            base = b * BKTN
            sv = startv[pl.ds(b, 16)]
            start = sv[0]
            n_e = sv[1] - sv[0]
            nwin = lax.shift_right_logical(n_e, 7)
            r0 = lax.shift_right_logical(start, 7)
            roff = lax.bitwise_and(r0, 7)
            r0a = pl.multiple_of(r0 - roff, 8)

            pltpu.sync_copy(pkg_hbm.at[pl.ds(r0a, MROWS + 8)], pk2d)
            pltpu.sync_copy(wgtg_hbm.at[pl.ds(r0a, MROWS + 8)], wg2d)

            def zero(d, _):
                for j in range(8):
                    sl = pl.ds(j * 16, 16)
                    s1a[d, sl] = z16
                    s2a[d, sl] = z16
                    mxa[d, sl] = lo16
                    mna[d, sl] = hi16
                dega[d] = jnp.float32(0.0)
                return 0
            lax.fori_loop(0, 136, zero, 0)

            iota16b = lax.broadcasted_iota(jnp.int32, (16,), 0)

            def build_srcidx(w):
                par = lax.bitwise_and(w, 1)
                for gq in range(8):
                    sl = pl.ds(gq * 16, 16)
                    srcidx[par, sl] = jnp.bitwise_and(pk2d[roff + w, sl],
                                                      16383)

            @pl.when(nwin > 0)
            def _prime():
                build_srcidx(0)
                pltpu.async_copy(b_hbm.at[srcidx.at[0]], rowsv.at[0], sem0)

            def window(w, _):
                par = lax.bitwise_and(w, 1)
                nxt = w + 1

                @pl.when((nxt < nwin) & (lax.bitwise_and(nxt, 1) == 0))
                def _fire0():
                    build_srcidx(nxt)
                    pltpu.async_copy(b_hbm.at[srcidx.at[0]],
                                     rowsv.at[0], sem0)

                @pl.when((nxt < nwin) & (lax.bitwise_and(nxt, 1) == 1))
                def _fire1():
                    build_srcidx(nxt)
                    pltpu.async_copy(b_hbm.at[srcidx.at[1]],
                                     rowsv.at[1], sem1)

                @pl.when(par == 0)
                def _wait0():
                    pltpu.make_async_copy(b_hbm.at[pl.ds(0, KW)],
                                          rowsv.at[0], sem0).wait()

                @pl.when(par == 1)
                def _wait1():
                    pltpu.make_async_copy(b_hbm.at[pl.ds(0, KW)],
                                          rowsv.at[1], sem1).wait()

                def group(gq, _):
                    sl16 = pl.ds(gq * 16, 16)
                    pvec = pk2d[roff + w, sl16]
                    dvec = lax.shift_right_logical(pvec, 16)
                    wvec = wg2d[roff + w, sl16]
                    tyvec = jnp.bitwise_and(lax.shift_right_logical(pvec, 14),
                                            3)
                    for k in range(16):
                        e = gq * 16 + k
                        d = dvec[k]
                        ty = tyvec[k]
                        wbc = wvec.at[jnp.full((16,), k, jnp.int32)].get(
                            mode="promise_in_bounds")
                        dega[d] = dega[d] + 1.0
                        for j in range(8):
                            sl = pl.ds(j * 16, 16)
                            r = rowsv[par, e, sl] + wbc * tv[ty, sl]
                            plsc.addupdate(s1a.at[d, sl], r)
                            plsc.addupdate(s2a.at[d, sl], r * r)
                            mxa[d, sl] = jnp.maximum(mxa[d, sl], r)
                            mna[d, sl] = jnp.minimum(mna[d, sl], r)
                    return 0
                lax.fori_loop(0, 8, group, 0)
                return 0
            lax.fori_loop(0, nwin, window, 0)

            def deg_tovmem(d, _):
                _sstore(degav, d, dega[d], jnp.float32)
                return 0
            lax.fori_loop(0, BKTN, deg_tovmem, 0)

            dsl = pl.ds(0, BKTN)
            osl = pl.ds(pl.multiple_of(base, BKTN), BKTN)
            pltpu.sync_copy(s1a.at[dsl], s1_hbm.at[osl])
            pltpu.sync_copy(s2a.at[dsl], s2_hbm.at[osl])
            pltpu.sync_copy(mxa.at[dsl], mx_hbm.at[osl])
            pltpu.sync_copy(mna.at[dsl], mn_hbm.at[osl])
            pltpu.sync_copy(degav.at[dsl], deg_hbm.at[osl])
        return 0
    lax.fori_loop(0, 3, do_round, 0)


def _mm_bias_body(x_ref, w_ref, b_ref, o_ref):
    o_ref[...] = x_ref[...] @ w_ref[...] + b_ref[...]


def _mm_bias(x, w, b, bm):
    m = x.shape[0]
    return pl.pallas_call(
        _mm_bias_body,
        grid=(m // bm,),
        in_specs=[pl.BlockSpec((bm, x.shape[1]), lambda i: (i, 0)),
                  pl.BlockSpec(w.shape, lambda i: (0, 0)),
                  pl.BlockSpec((1, w.shape[1]), lambda i: (0, 0))],
        out_specs=pl.BlockSpec((bm, w.shape[1]), lambda i: (i, 0)),
        out_shape=jax.ShapeDtypeStruct((m, w.shape[1]), jnp.float32),
    )(x, w, b[None, :])


def _ab_body(x_ref, wpre_ref, wedge_ref, eemb_ref, bedge_ref, bpre_ref,
             a_ref, b_ref, t_ref):
    w1 = wpre_ref[0:H, :]
    w2 = wpre_ref[H:2 * H, :]
    w3 = wpre_ref[2 * H:3 * H, :]
    x = x_ref[...]
    a_ref[...] = x @ w1
    c0 = bedge_ref[...] @ w3 + bpre_ref[...]
    b_ref[...] = x @ w2 + c0
    t_ref[...] = (eemb_ref[...] @ wedge_ref[...]) @ w3


def _ab_stage(entity, lp, eemb):
    bm = 1000
    return pl.pallas_call(
        _ab_body,
        grid=(N_NODES // bm,),
        in_specs=[pl.BlockSpec((bm, H), lambda i: (i, 0)),
                  pl.BlockSpec((3 * H, H), lambda i: (0, 0)),
                  pl.BlockSpec((16, H), lambda i: (0, 0)),
                  pl.BlockSpec((4, 16), lambda i: (0, 0)),
                  pl.BlockSpec((1, H), lambda i: (0, 0)),
                  pl.BlockSpec((1, H), lambda i: (0, 0))],
        out_specs=(pl.BlockSpec((bm, H), lambda i: (i, 0)),
                   pl.BlockSpec((bm, H), lambda i: (i, 0)),
                   pl.BlockSpec((4, H), lambda i: (0, 0))),
        out_shape=(jax.ShapeDtypeStruct((N_NODES, H), jnp.float32),
                   jax.ShapeDtypeStruct((N_NODES, H), jnp.float32),
                   jax.ShapeDtypeStruct((4, H), jnp.float32)),
    )(entity, lp['W_pre'], lp['W_edge'], eemb, lp['b_edge'][None, :],
      lp['b_pre'][None, :])


def _post_body(x_ref, a_ref, s1_ref, s2_ref, mx_ref, mn_ref, deg_ref,
               adl_ref, wpost_ref, bpost_ref, wlin_ref, blin_ref, o_ref):
    deg = deg_ref[...]
    degc = jnp.maximum(deg, 1.0)
    idegc = 1.0 / degc
    has = deg > 0
    a = a_ref[...]
    s1n = s1_ref[...] * idegc
    mean = jnp.where(has, a + s1n, 0.0)
    std = jnp.sqrt(jax.nn.relu(s2_ref[...] * idegc - s1n * s1n) + 1e-5)
    mx = jnp.where(has, a + mx_ref[...], 0.0)
    mn = jnp.where(has, a + mn_ref[...], 0.0)
    agg = jnp.concatenate([mean, mn, mx, std], axis=-1)
    adl = adl_ref[0, 0]
    lg = jnp.log(degc + 1.0)
    amp = lg * (1.0 / adl)
    att = adl / lg
    cat = jnp.concatenate([x_ref[...], agg, agg * amp, agg * att], axis=-1)
    out = cat @ wpost_ref[...] + bpost_ref[...]
    out = out @ wlin_ref[...] + blin_ref[...]
    o_ref[...] = _leaky(out)


def _post_stage(entity, a, s1, s2, mx, mn, deg, lp):
    bm = 1000
    nb = N_NODES // bm
    row = lambda i: (i, 0)
    fixed = lambda i: (0, 0)
    return pl.pallas_call(
        _post_body,
        grid=(nb,),
        in_specs=[pl.BlockSpec((bm, H), row),
                  pl.BlockSpec((bm, H), row),
                  pl.BlockSpec((bm, H), row),
                  pl.BlockSpec((bm, H), row),
                  pl.BlockSpec((bm, H), row),
                  pl.BlockSpec((bm, H), row),
                  pl.BlockSpec((bm, 1), row),
                  pl.BlockSpec((1, 1), fixed),
                  pl.BlockSpec((13 * H, H), fixed),
                  pl.BlockSpec((1, H), fixed),
                  pl.BlockSpec((H, H), fixed),
                  pl.BlockSpec((1, H), fixed)],
        out_specs=pl.BlockSpec((bm, H), row),
        out_shape=jax.ShapeDtypeStruct((N_NODES, H), jnp.float32),
    )(entity, a, s1, s2, mx, mn, deg[:, None],
      jnp.reshape(lp['avg_deg_log'], (1, 1)), lp['W_post'],
      lp['b_post'][None, :], lp['W_lin'], lp['b_lin'][None, :])


def _heads_body(comp_ref, prot_ref, wfc_ref, bfc_ref, wfp_ref, bfp_ref,
                w0_ref, w1_ref, u_ref, v_ref):
    fc = _leaky(comp_ref[...] @ wfc_ref[...] + bfc_ref[...])
    fc = fc / jnp.clip(jnp.sqrt(jnp.sum(fc * fc, axis=1, keepdims=True)), 1e-12)
    u_ref[...] = fc @ w0_ref[...]
    fp = _leaky(prot_ref[...] @ wfp_ref[...] + bfp_ref[...])
    fp = fp / jnp.clip(jnp.sqrt(jnp.sum(fp * fp, axis=1, keepdims=True)), 1e-12)
    v_ref[...] = fp @ w1_ref[...]


def _final_stage(comp, prot, params):
    return pl.pallas_call(
        _heads_body,
        out_shape=(jax.ShapeDtypeStruct((N_COMPOUND, 64), jnp.float32),
                   jax.ShapeDtypeStruct((N_PROTEIN, 64), jnp.float32)),
    )(comp, prot, params['Wfc'], params['bfc'][None, :], params['Wfp'],
      params['bfp'][None, :], params['W0p'], params['W1p'])


def _recon_body(u_ref, v_ref, cpi_ref, mask_ref, recon_ref, loss_ref):
    i = pl.program_id(0)
    j = pl.program_id(1)
    recon = u_ref[...] @ v_ref[...].T
    recon_ref[...] = recon
    tmp = mask_ref[...] * (recon - cpi_ref[...])
    part = jnp.sum(tmp * tmp)

    @pl.when((i == 0) & (j == 0))
    def _init():
        loss_ref[0, 0] = 0.0

    loss_ref[0, 0] += part


def _recon_loss(u, v, CPI, CPI_mask):
    bm, bn = 280, 3000
    grid = (N_COMPOUND // bm, N_PROTEIN // bn)
    recon, loss = pl.pallas_call(
        _recon_body,
        grid=grid,
        in_specs=[
            pl.BlockSpec((bm, 64), lambda i, j: (i, 0)),
            pl.BlockSpec((bn, 64), lambda i, j: (j, 0)),
            pl.BlockSpec((bm, bn), lambda i, j: (i, j)),
            pl.BlockSpec((bm, bn), lambda i, j: (i, j)),
        ],
        out_specs=(pl.BlockSpec((bm, bn), lambda i, j: (i, j)),
                   pl.BlockSpec((1, 1), lambda i, j: (0, 0),
                                memory_space=pltpu.SMEM)),
        out_shape=(jax.ShapeDtypeStruct((N_COMPOUND, N_PROTEIN), jnp.float32),
                   jax.ShapeDtypeStruct((1, 1), jnp.float32)),
    )(u, v, CPI, CPI_mask)
    return loss[0, 0], recon


def kernel(compound_embedding, protein_embedding, edge_index, edge_weight,
           edge_type, CPI, CPI_mask, params):
    src = edge_index[0]
    dst = edge_index[1]
    ew = edge_weight[:, 0]
    etype = edge_type.astype(jnp.int32)

    compound = _mm_bias(compound_embedding, params['Wc'], params['bc'], 1000)
    protein = _mm_bias(protein_embedding, params['Wp'], params['bp'], 1000)
    entity = jnp.concatenate([compound, protein], axis=0)
    entity0 = entity

    cnt = _sc_histogram(dst)
    packed_g, wgt_g, bkt_start = _sc_permute(src, ew, etype, dst, cnt)
    packed_g = packed_g.reshape(NR, 128)
    wgt_g = wgt_g.reshape(NR, 128)

    for lp in params['layers']:
        a, b_eff, t_tab = _ab_stage(entity, lp, params['edge_emb'])
        s1, s2, mx, mn, deg = _sc_stats(b_eff, t_tab, packed_g, wgt_g,
                                        bkt_start)
        entity = _post_stage(entity, a, s1[:N_NODES], s2[:N_NODES],
                             mx[:N_NODES], mn[:N_NODES], deg[:N_NODES], lp)

    entity = jnp.concatenate([entity, entity0], axis=1)
    comp = entity[:N_COMPOUND]
    prot = entity[N_COMPOUND:N_COMPOUND + N_PROTEIN]
    u, v = _final_stage(comp, prot, params)
    return _recon_loss(u, v, CPI, CPI_mask)
